# baseline (speedup 1.0000x reference)
import math
import functools

import jax
import jax.numpy as jnp
from jax import lax
from jax.experimental import pallas as pl
from jax.experimental.pallas import tpu as pltpu
from jax.experimental.pallas import tpu_sc as plsc

SIZE_ = 32000
PAD_ = 0
SMOOTH_ = 0.1
CONF_ = 1.0 - SMOOTH_
SVAL_ = SMOOTH_ / (SIZE_ - 2)
K_ = (SIZE_ - 2) * SVAL_ * math.log(SVAL_) + CONF_ * math.log(CONF_)

B_ = 2048
BR_ = 128

_NC = 2
_NS = 16
_NW = _NC * _NS
_TPW = B_ // _NW
_L = 16


def _tc_body(out_ref, t_ref, sc_ref, acc_ref):
    j = pl.program_id(0)
    out_blk = out_ref[...]
    t_blk = t_ref[...]
    nonpad = t_blk != PAD_

    colids = lax.broadcasted_iota(jnp.int32, (BR_, SIZE_), 1)
    rowsum = jnp.sum(out_blk, axis=1, keepdims=True)
    pick = jnp.sum(jnp.where(colids == t_blk, out_blk, 0.0),
                   axis=1, keepdims=True)
    out0 = out_blk[:, 0:1]
    per_row = -SVAL_ * (rowsum - out0) - (CONF_ - SVAL_) * pick
    partial = jnp.sum(jnp.where(nonpad, per_row, 0.0))

    @pl.when(j == 0)
    def _init():
        acc_ref[0, 0] = jnp.sum(sc_ref[...])

    acc_ref[0, 0] = acc_ref[0, 0] + partial


def _sc_body(tgt_hbm, part_hbm, tgt_v, part_v):
    wid = lax.axis_index("s") * _NC + lax.axis_index("c")
    base = wid * _TPW
    pltpu.sync_copy(tgt_hbm.at[pl.ds(base, _TPW)], tgt_v)
    acc = jnp.zeros((_L,), jnp.float32)
    for kk in range(_TPW // _L):
        t16 = tgt_v[pl.ds(kk * _L, _L)]
        acc = acc + jnp.where(t16 != PAD_, K_, 0.0)
    part_v[...] = acc
    pltpu.sync_copy(part_v, part_hbm.at[wid])


_sc_entropy = functools.partial(
    pl.kernel,
    mesh=plsc.VectorSubcoreMesh(core_axis_name="c", subcore_axis_name="s"),
    out_type=jax.ShapeDtypeStruct((_NW, _L), jnp.float32),
    scratch_types=[
        pltpu.VMEM((_TPW,), jnp.int32),
        pltpu.VMEM((_L,), jnp.float32),
    ],
)(_sc_body)


@jax.jit
def kernel(output, target, one_hot):
    del one_hot
    t32 = target.astype(jnp.int32)
    sc_parts = _sc_entropy(t32)

    t2 = t32.reshape(B_, 1)
    acc = pl.pallas_call(
        _tc_body,
        grid=(B_ // BR_,),
        in_specs=[
            pl.BlockSpec((BR_, SIZE_), lambda j: (j, 0)),
            pl.BlockSpec((BR_, 1), lambda j: (j, 0)),
            pl.BlockSpec((_NW, _L), lambda j: (0, 0)),
        ],
        out_specs=pl.BlockSpec(
            (1, 1), lambda j: (0, 0), memory_space=pltpu.SMEM
        ),
        out_shape=jax.ShapeDtypeStruct((1, 1), jnp.float32),
    )(output, t2, sc_parts)
    return acc[0, 0]


# speedup vs baseline: 1.0265x; 1.0265x over previous
import math
import functools

import jax
import jax.numpy as jnp
from jax import lax
from jax.experimental import pallas as pl
from jax.experimental.pallas import tpu as pltpu
from jax.experimental.pallas import tpu_sc as plsc

SIZE_ = 32000
PAD_ = 0
SMOOTH_ = 0.1
CONF_ = 1.0 - SMOOTH_
SVAL_ = SMOOTH_ / (SIZE_ - 2)
K_ = (SIZE_ - 2) * SVAL_ * math.log(SVAL_) + CONF_ * math.log(CONF_)

B_ = 2048
BR_ = 128

_NC = 1
_NS = 16
_NW = _NC * _NS
_TPW = B_ // _NW
_L = 16


def _tc_body(out_ref, t_ref, acc_ref):
    j = pl.program_id(0)
    out_blk = out_ref[...]
    t_blk = t_ref[...]
    nonpad = t_blk != PAD_

    colids = lax.broadcasted_iota(jnp.int32, (BR_, SIZE_), 1)
    rowsum = jnp.sum(out_blk, axis=1, keepdims=True)
    pick = jnp.sum(jnp.where(colids == t_blk, out_blk, 0.0),
                   axis=1, keepdims=True)
    out0 = out_blk[:, 0:1]
    per_row = -SVAL_ * (rowsum - out0) - (CONF_ - SVAL_) * pick
    partial = jnp.sum(jnp.where(nonpad, per_row, 0.0))

    @pl.when(j == 0)
    def _init():
        acc_ref[0, 0] = 0.0

    acc_ref[0, 0] = acc_ref[0, 0] + partial


def _sc_body(tgt_hbm, part_hbm, tgt_v, part_v):
    wid = lax.axis_index("s") * _NC + lax.axis_index("c")
    base = wid * _TPW
    pltpu.sync_copy(tgt_hbm.at[pl.ds(base, _TPW)], tgt_v)
    acc = jnp.zeros((_L,), jnp.float32)
    for kk in range(_TPW // _L):
        t16 = tgt_v[pl.ds(kk * _L, _L)]
        acc = acc + jnp.where(t16 != PAD_, K_, 0.0)
    part_v[...] = acc
    pltpu.sync_copy(part_v, part_hbm.at[wid])


_sc_entropy = functools.partial(
    pl.kernel,
    mesh=plsc.VectorSubcoreMesh(core_axis_name="c", subcore_axis_name="s", num_cores=1),
    out_type=jax.ShapeDtypeStruct((_NW, _L), jnp.float32),
    scratch_types=[
        pltpu.VMEM((_TPW,), jnp.int32),
        pltpu.VMEM((_L,), jnp.float32),
    ],
)(_sc_body)


@jax.jit
def kernel(output, target, one_hot):
    del one_hot
    t32 = target.astype(jnp.int32)
    sc_parts = _sc_entropy(t32)

    t2 = t32.reshape(B_, 1)
    acc = pl.pallas_call(
        _tc_body,
        grid=(B_ // BR_,),
        in_specs=[
            pl.BlockSpec((BR_, SIZE_), lambda j: (j, 0)),
            pl.BlockSpec((BR_, 1), lambda j: (j, 0)),
        ],
        out_specs=pl.BlockSpec(
            (1, 1), lambda j: (0, 0), memory_space=pltpu.SMEM
        ),
        out_shape=jax.ShapeDtypeStruct((1, 1), jnp.float32),
    )(output, t2)
    return acc[0, 0] + jnp.sum(sc_parts)


# speedup vs baseline: 1.2435x; 1.2114x over previous
import math

import jax
import jax.numpy as jnp
from jax import lax
from jax.experimental import pallas as pl
from jax.experimental.pallas import tpu as pltpu

SIZE_ = 32000
PAD_ = 0
SMOOTH_ = 0.1
CONF_ = 1.0 - SMOOTH_
SVAL_ = SMOOTH_ / (SIZE_ - 2)
K_ = (SIZE_ - 2) * SVAL_ * math.log(SVAL_) + CONF_ * math.log(CONF_)

B_ = 2048
BR_ = 128


def _loss_body(out_ref, t_ref, acc_ref):
    j = pl.program_id(0)
    out_blk = out_ref[...]
    t_blk = t_ref[...]
    nonpad = t_blk != PAD_

    colids = lax.broadcasted_iota(jnp.int32, (BR_, SIZE_), 1)
    rowsum = jnp.sum(out_blk, axis=1, keepdims=True)
    pick = jnp.sum(jnp.where(colids == t_blk, out_blk, 0.0),
                   axis=1, keepdims=True)
    out0 = out_blk[:, 0:1]
    per_row = K_ - SVAL_ * (rowsum - out0) - (CONF_ - SVAL_) * pick
    partial = jnp.sum(jnp.where(nonpad, per_row, 0.0))

    @pl.when(j == 0)
    def _init():
        acc_ref[0, 0] = 0.0

    acc_ref[0, 0] = acc_ref[0, 0] + partial


@jax.jit
def kernel(output, target, one_hot):
    del one_hot
    t2 = target.astype(jnp.int32).reshape(B_, 1)
    acc = pl.pallas_call(
        _loss_body,
        grid=(B_ // BR_,),
        in_specs=[
            pl.BlockSpec((BR_, SIZE_), lambda j: (j, 0)),
            pl.BlockSpec((BR_, 1), lambda j: (j, 0)),
        ],
        out_specs=pl.BlockSpec(
            (1, 1), lambda j: (0, 0), memory_space=pltpu.SMEM
        ),
        out_shape=jax.ShapeDtypeStruct((1, 1), jnp.float32),
    )(output, t2)
    return acc[0, 0]
